# baseline (speedup 1.0000x reference)
import math

import jax
import jax.numpy as jnp
from jax import lax
from jax.experimental import pallas as pl
from jax.experimental.pallas import tpu as pltpu
from jax.experimental.pallas import tpu_sc as plsc

N_NODES = 10000
N_EDGES = 160000
D_FEAT = 128
D_EDGE = 4
D_RAD = 8
RADIAL_HIDDEN = 8
D_MSG = D_FEAT + D_FEAT * D_EDGE
AVG_NEIGHBOURS = 16.0

_INV_SQRT_RAD = 1.0 / math.sqrt(float(D_RAD))
_W2_SCALE = (1.0 / math.sqrt(float(RADIAL_HIDDEN))) / math.sqrt(AVG_NEIGHBOURS)

NC = 2
NS = 16
NW = NC * NS

EB = 128
E_PAD = 163840
N_EBLK = E_PAD // EB
N_CHUNK = 5
D_CHUNK = 128

NODE_TILE = 624
NODE_TAIL = N_NODES - NODE_TILE * NS
ZCHUNK = 48

EDGE_BLOCK = 2048


def _sc_mesh():
    return plsc.VectorSubcoreMesh(core_axis_name="c", subcore_axis_name="s")


GDEPTH = 5


def _gather_body(nf_hbm, snd_hbm, out_hbm, idx_v, rows_v, *sems):
    c = lax.axis_index("c")
    s = lax.axis_index("s")
    wid = s * NC + c
    nblk = N_EBLK // NW
    row0 = wid * nblk
    pltpu.sync_copy(snd_hbm.at[pl.ds(row0, nblk)], idx_v)

    for par in range(GDEPTH):
        pltpu.make_async_copy(
            nf_hbm.at[idx_v.at[par, 0]], rows_v.at[par], sems[par]
        ).start()

    def body(g, carry):
        for par in range(GDEPTH):
            b = g * GDEPTH + par
            pltpu.make_async_copy(
                nf_hbm.at[idx_v.at[b, 0]], rows_v.at[par], sems[par]
            ).wait()
            pltpu.sync_copy(rows_v.at[par], out_hbm.at[pl.ds((row0 + b) * EB, EB)])

            @pl.when(g < nblk // GDEPTH - 1)
            def _next():
                pltpu.make_async_copy(
                    nf_hbm.at[idx_v.at[b + GDEPTH, 0]], rows_v.at[par], sems[par]
                ).start()

        return carry

    lax.fori_loop(0, nblk // GDEPTH, body, 0)


def _sc_gather(node_feats, senders_3d):
    return pl.kernel(
        _gather_body,
        out_type=jax.ShapeDtypeStruct((E_PAD, D_FEAT), jnp.float32),
        mesh=_sc_mesh(),
        scratch_types=[
            pltpu.VMEM((N_EBLK // NW, 1, EB), jnp.int32),
            pltpu.VMEM((GDEPTH, EB, D_FEAT), jnp.float32),
        ]
        + [pltpu.SemaphoreType.DMA] * GDEPTH,
    )(node_feats, senders_3d)


def _make_dense_body(chunks):
    nch = len(chunks)

    def _dense_body(msgs_ref, ef_ref, rad_ref, w1_ref, w2s_ref, out_ref):
        rad = rad_ref[...]
        h = jnp.dot(rad, w1_ref[...], preferred_element_type=jnp.float32)
        h = h * _INV_SQRT_RAD
        h = h * jax.nn.sigmoid(h)
        w = jnp.dot(h, w2s_ref[...], preferred_element_type=jnp.float32)
        w = w * _W2_SCALE
        m = msgs_ref[...]
        ef = ef_ref[...]
        for ci, ch in enumerate(chunks):
            wpart = w[:, ci * D_FEAT : (ci + 1) * D_FEAT]
            if ch == 0:
                out_ref[ci] = m * wpart
            else:
                out_ref[ci] = m * ef[:, ch - 1 : ch] * wpart

    return _dense_body


def _dense_messages(msgs, edge_features, radial_embedding, W1, W2r, chunks):
    w2s = jnp.concatenate(
        [W2r[:, ch * D_FEAT : (ch + 1) * D_FEAT] for ch in chunks], axis=1
    )
    nch = len(chunks)
    grid = (E_PAD // EDGE_BLOCK,)
    return pl.pallas_call(
        _make_dense_body(chunks),
        grid=grid,
        in_specs=[
            pl.BlockSpec((EDGE_BLOCK, D_FEAT), lambda i: (i, 0)),
            pl.BlockSpec((EDGE_BLOCK, D_EDGE), lambda i: (i, 0)),
            pl.BlockSpec((EDGE_BLOCK, D_RAD), lambda i: (i, 0)),
            pl.BlockSpec((D_RAD, RADIAL_HIDDEN), lambda i: (0, 0)),
            pl.BlockSpec((RADIAL_HIDDEN, nch * D_FEAT), lambda i: (0, 0)),
        ],
        out_specs=pl.BlockSpec((nch, EDGE_BLOCK, D_CHUNK), lambda i: (0, i, 0)),
        out_shape=jax.ShapeDtypeStruct((nch, E_PAD, D_CHUNK), jnp.float32),
    )(msgs, edge_features, radial_embedding, W1, w2s)


def _make_scatter_body(n_full, has_half):
    def _scatter_body(msg_hbm, recv_hbm, out_hbm, idx_v, buf_v, zbuf_v, acc_sh, sem0, sem1):
        c = lax.axis_index("c")
        s = lax.axis_index("s")
        nblk_full = N_EBLK // NS
        r0 = s * nblk_full
        sems = (sem0, sem1)

        def zero_body(i, carry):
            r = i // (D_CHUNK // 16)
            g = i % (D_CHUNK // 16)
            zbuf_v[r, pl.ds(g * 16, 16)] = jnp.zeros((16,), jnp.float32)
            return carry

        lax.fori_loop(0, ZCHUNK * (D_CHUNK // 16), zero_body, 0)

        pltpu.sync_copy(recv_hbm.at[pl.ds(r0, nblk_full)], idx_v)

        n_passes = n_full + (1 if has_half else 0)
        for p in range(n_passes):
            if p < n_full:
                chunk = c * n_full + p
                slot = chunk
                b_lo = 0
                nblk = nblk_full
            else:
                chunk = 2 * n_full
                slot = 2 * n_full + c
                b_lo = c * (nblk_full // 2)
                nblk = nblk_full // 2

            for k in range(NODE_TILE // ZCHUNK):
                pltpu.sync_copy(zbuf_v, acc_sh.at[pl.ds(s * NODE_TILE + k * ZCHUNK, ZCHUNK)])

            @pl.when(s == 0)
            def _zero_tail():
                pltpu.sync_copy(
                    zbuf_v.at[pl.ds(0, NODE_TAIL)],
                    acc_sh.at[pl.ds(NODE_TILE * NS, NODE_TAIL)],
                )

            plsc.subcore_barrier()

            for par in range(2):
                pltpu.make_async_copy(
                    msg_hbm.at[chunk, pl.ds((r0 + b_lo + par) * EB, EB)],
                    buf_v.at[par],
                    sems[par],
                ).start()

            def acc_body(g, carry):
                for par in range(2):
                    lb = b_lo + g * 2 + par
                    pltpu.make_async_copy(
                        msg_hbm.at[chunk, pl.ds((r0 + lb) * EB, EB)],
                        buf_v.at[par],
                        sems[par],
                    ).wait()
                    pltpu.sync_copy(buf_v.at[par], acc_sh.at[idx_v.at[lb, 0]], add=True)

                    @pl.when(g < nblk // 2 - 1)
                    def _next():
                        pltpu.make_async_copy(
                            msg_hbm.at[chunk, pl.ds((r0 + lb + 2) * EB, EB)],
                            buf_v.at[par],
                            sems[par],
                        ).start()

                return carry

            lax.fori_loop(0, nblk // 2, acc_body, 0)
            plsc.subcore_barrier()

            off = s * NODE_TILE
            pltpu.sync_copy(acc_sh.at[pl.ds(off, NODE_TILE)], out_hbm.at[slot, pl.ds(off, NODE_TILE)])

            @pl.when(s == 0)
            def _flush_tail():
                off_t = NODE_TILE * NS
                pltpu.sync_copy(
                    acc_sh.at[pl.ds(off_t, NODE_TAIL)],
                    out_hbm.at[slot, pl.ds(off_t, NODE_TAIL)],
                )

            plsc.subcore_barrier()

    return _scatter_body


def _sc_scatter(msg_parts, receivers_3d, n_full, has_half):
    n_parts = msg_parts.shape[0]
    n_slots = 2 * n_full + (2 if has_half else 0)
    assert n_parts == 2 * n_full + (1 if has_half else 0)
    return pl.kernel(
        _make_scatter_body(n_full, has_half),
        out_type=jax.ShapeDtypeStruct((n_slots, N_NODES, D_CHUNK), jnp.float32),
        mesh=_sc_mesh(),
        scratch_types=[
            pltpu.VMEM((N_EBLK // NS, 1, EB), jnp.int32),
            pltpu.VMEM((2, EB, D_CHUNK), jnp.float32),
            pltpu.VMEM((ZCHUNK, D_CHUNK), jnp.float32),
            pltpu.VMEM_SHARED((N_NODES, D_CHUNK), jnp.float32),
            pltpu.SemaphoreType.DMA,
            pltpu.SemaphoreType.DMA,
        ],
    )(msg_parts, receivers_3d)


def kernel(node_feats, edge_features, radial_embedding, senders, receivers, W1, W2):
    pad = E_PAD - N_EDGES
    senders_3d = jnp.pad(senders.astype(jnp.int32), (0, pad)).reshape(N_EBLK, 1, EB)
    receivers_3d = jnp.pad(receivers.astype(jnp.int32), (0, pad)).reshape(N_EBLK, 1, EB)
    ef_p = jnp.pad(edge_features, ((0, pad), (0, 0)))
    rad_p = jnp.pad(radial_embedding, ((0, pad), (0, 0)))
    W2r = jnp.concatenate(
        [
            W2[:, :D_FEAT],
            W2[:, D_FEAT:]
            .reshape(RADIAL_HIDDEN, D_FEAT, D_EDGE)
            .transpose(0, 2, 1)
            .reshape(RADIAL_HIDDEN, D_FEAT * D_EDGE),
        ],
        axis=1,
    )
    msgs = _sc_gather(node_feats, senders_3d)
    msgA = _dense_messages(msgs, ef_p, rad_p, W1, W2r, chunks=(0, 2))
    msgB = _dense_messages(msgs, ef_p, rad_p, W1, W2r, chunks=(1, 3, 4))
    outA = _sc_scatter(msgA, receivers_3d, n_full=1, has_half=False)
    outB = _sc_scatter(msgB, receivers_3d, n_full=1, has_half=True)
    blocked = jnp.concatenate(
        [outA[0], outB[0], outA[1], outB[1], outB[2] + outB[3]], axis=-1
    )
    direct = blocked[:, :D_FEAT]
    tp = (
        blocked[:, D_FEAT:]
        .reshape(N_NODES, D_EDGE, D_FEAT)
        .transpose(0, 2, 1)
        .reshape(N_NODES, D_FEAT * D_EDGE)
    )
    return jnp.concatenate([direct, tp], axis=-1)


# speedup vs baseline: 1.0002x; 1.0002x over previous
import math

import jax
import jax.numpy as jnp
from jax import lax
from jax.experimental import pallas as pl
from jax.experimental.pallas import tpu as pltpu
from jax.experimental.pallas import tpu_sc as plsc

N_NODES = 10000
N_EDGES = 160000
D_FEAT = 128
D_EDGE = 4
D_RAD = 8
RADIAL_HIDDEN = 8
D_MSG = D_FEAT + D_FEAT * D_EDGE
AVG_NEIGHBOURS = 16.0

_INV_SQRT_RAD = 1.0 / math.sqrt(float(D_RAD))
_W2_SCALE = (1.0 / math.sqrt(float(RADIAL_HIDDEN))) / math.sqrt(AVG_NEIGHBOURS)

NC = 2
NS = 16
NW = NC * NS

EB = 128
E_PAD = 163840
N_EBLK = E_PAD // EB
N_CHUNK = 5
D_CHUNK = 128

NODE_TILE = 624
NODE_TAIL = N_NODES - NODE_TILE * NS
ZCHUNK = 48

EDGE_BLOCK = 2048


def _sc_mesh():
    return plsc.VectorSubcoreMesh(core_axis_name="c", subcore_axis_name="s")


GDEPTH = 5


def _gather_body(nf_hbm, snd_hbm, out_hbm, idx_v, rows_v, *sems):
    c = lax.axis_index("c")
    s = lax.axis_index("s")
    wid = s * NC + c
    nblk = N_EBLK // NW
    row0 = wid * nblk
    pltpu.sync_copy(snd_hbm.at[pl.ds(row0, nblk)], idx_v)

    for par in range(GDEPTH):
        pltpu.make_async_copy(
            nf_hbm.at[idx_v.at[par, 0]], rows_v.at[par], sems[par]
        ).start()

    def body(g, carry):
        for par in range(GDEPTH):
            b = g * GDEPTH + par
            pltpu.make_async_copy(
                nf_hbm.at[idx_v.at[b, 0]], rows_v.at[par], sems[par]
            ).wait()
            pltpu.sync_copy(rows_v.at[par], out_hbm.at[pl.ds((row0 + b) * EB, EB)])

            @pl.when(g < nblk // GDEPTH - 1)
            def _next():
                pltpu.make_async_copy(
                    nf_hbm.at[idx_v.at[b + GDEPTH, 0]], rows_v.at[par], sems[par]
                ).start()

        return carry

    lax.fori_loop(0, nblk // GDEPTH, body, 0)


def _sc_gather(node_feats, senders_3d):
    return pl.kernel(
        _gather_body,
        out_type=jax.ShapeDtypeStruct((E_PAD, D_FEAT), jnp.float32),
        mesh=_sc_mesh(),
        scratch_types=[
            pltpu.VMEM((N_EBLK // NW, 1, EB), jnp.int32),
            pltpu.VMEM((GDEPTH, EB, D_FEAT), jnp.float32),
        ]
        + [pltpu.SemaphoreType.DMA] * GDEPTH,
    )(node_feats, senders_3d)


def _make_dense_body(chunks):
    nch = len(chunks)

    def _dense_body(msgs_ref, ef_ref, rad_ref, w1_ref, w2s_ref, out_ref):
        rad = rad_ref[...]
        h = jnp.dot(rad, w1_ref[...], preferred_element_type=jnp.float32)
        h = h * _INV_SQRT_RAD
        h = h * jax.nn.sigmoid(h)
        w = jnp.dot(h, w2s_ref[...], preferred_element_type=jnp.float32)
        w = w * _W2_SCALE
        m = msgs_ref[...]
        ef = ef_ref[...]
        for ci, ch in enumerate(chunks):
            wpart = w[:, ci * D_FEAT : (ci + 1) * D_FEAT]
            if ch == 0:
                out_ref[ci] = m * wpart
            else:
                out_ref[ci] = m * ef[:, ch - 1 : ch] * wpart

    return _dense_body


def _dense_messages(msgs, edge_features, radial_embedding, W1, W2r, chunks):
    w2s = jnp.concatenate(
        [W2r[:, ch * D_FEAT : (ch + 1) * D_FEAT] for ch in chunks], axis=1
    )
    nch = len(chunks)
    grid = (E_PAD // EDGE_BLOCK,)
    return pl.pallas_call(
        _make_dense_body(chunks),
        grid=grid,
        in_specs=[
            pl.BlockSpec((EDGE_BLOCK, D_FEAT), lambda i: (i, 0)),
            pl.BlockSpec((EDGE_BLOCK, D_EDGE), lambda i: (i, 0)),
            pl.BlockSpec((EDGE_BLOCK, D_RAD), lambda i: (i, 0)),
            pl.BlockSpec((D_RAD, RADIAL_HIDDEN), lambda i: (0, 0)),
            pl.BlockSpec((RADIAL_HIDDEN, nch * D_FEAT), lambda i: (0, 0)),
        ],
        out_specs=pl.BlockSpec((nch, EDGE_BLOCK, D_CHUNK), lambda i: (0, i, 0)),
        out_shape=jax.ShapeDtypeStruct((nch, E_PAD, D_CHUNK), jnp.float32),
    )(msgs, edge_features, radial_embedding, W1, w2s)


def _make_scatter_body(n_full, has_half):
    def _scatter_body(msg_hbm, recv_hbm, out_hbm, idx_v, buf_v, zbuf_v, acc_sh, sem0, sem1):
        c = lax.axis_index("c")
        s = lax.axis_index("s")
        nblk_full = N_EBLK // NS
        r0 = s * nblk_full
        sems = (sem0, sem1)

        def zero_body(i, carry):
            r = i // (D_CHUNK // 16)
            g = i % (D_CHUNK // 16)
            zbuf_v[r, pl.ds(g * 16, 16)] = jnp.zeros((16,), jnp.float32)
            return carry

        lax.fori_loop(0, ZCHUNK * (D_CHUNK // 16), zero_body, 0)

        pltpu.sync_copy(recv_hbm.at[pl.ds(r0, nblk_full)], idx_v)

        n_passes = n_full + (1 if has_half else 0)
        for p in range(n_passes):
            if p < n_full:
                chunk = c * n_full + p
                slot = chunk
                b_lo = 0
                nblk = nblk_full
            else:
                chunk = 2 * n_full
                slot = 2 * n_full + c
                b_lo = c * (nblk_full // 2)
                nblk = nblk_full // 2

            for k in range(NODE_TILE // ZCHUNK):
                pltpu.sync_copy(zbuf_v, acc_sh.at[pl.ds(s * NODE_TILE + k * ZCHUNK, ZCHUNK)])

            @pl.when(s == 0)
            def _zero_tail():
                pltpu.sync_copy(
                    zbuf_v.at[pl.ds(0, NODE_TAIL)],
                    acc_sh.at[pl.ds(NODE_TILE * NS, NODE_TAIL)],
                )

            plsc.subcore_barrier()

            for par in range(2):
                pltpu.make_async_copy(
                    msg_hbm.at[chunk, pl.ds((r0 + b_lo + par) * EB, EB)],
                    buf_v.at[par],
                    sems[par],
                ).start()

            def acc_body(g, carry):
                for par in range(2):
                    lb = b_lo + g * 2 + par
                    pltpu.make_async_copy(
                        msg_hbm.at[chunk, pl.ds((r0 + lb) * EB, EB)],
                        buf_v.at[par],
                        sems[par],
                    ).wait()
                    pltpu.sync_copy(buf_v.at[par], acc_sh.at[idx_v.at[lb, 0]], add=True)

                    @pl.when(g < nblk // 2 - 1)
                    def _next():
                        pltpu.make_async_copy(
                            msg_hbm.at[chunk, pl.ds((r0 + lb + 2) * EB, EB)],
                            buf_v.at[par],
                            sems[par],
                        ).start()

                return carry

            lax.fori_loop(0, nblk // 2, acc_body, 0)
            plsc.subcore_barrier()

            off = s * NODE_TILE
            pltpu.sync_copy(acc_sh.at[pl.ds(off, NODE_TILE)], out_hbm.at[slot, pl.ds(off, NODE_TILE)])

            @pl.when(s == 0)
            def _flush_tail():
                off_t = NODE_TILE * NS
                pltpu.sync_copy(
                    acc_sh.at[pl.ds(off_t, NODE_TAIL)],
                    out_hbm.at[slot, pl.ds(off_t, NODE_TAIL)],
                )

            plsc.subcore_barrier()

    return _scatter_body


def _sc_scatter(msg_parts, receivers_3d, n_full, has_half):
    n_parts = msg_parts.shape[0]
    n_slots = 2 * n_full + (2 if has_half else 0)
    assert n_parts == 2 * n_full + (1 if has_half else 0)
    return pl.kernel(
        _make_scatter_body(n_full, has_half),
        out_type=jax.ShapeDtypeStruct((n_slots, N_NODES, D_CHUNK), jnp.float32),
        mesh=_sc_mesh(),
        scratch_types=[
            pltpu.VMEM((N_EBLK // NS, 1, EB), jnp.int32),
            pltpu.VMEM((2, EB, D_CHUNK), jnp.float32),
            pltpu.VMEM((ZCHUNK, D_CHUNK), jnp.float32),
            pltpu.VMEM_SHARED((N_NODES, D_CHUNK), jnp.float32),
            pltpu.SemaphoreType.DMA,
            pltpu.SemaphoreType.DMA,
        ],
    )(msg_parts, receivers_3d)


def kernel(node_feats, edge_features, radial_embedding, senders, receivers, W1, W2):
    pad = E_PAD - N_EDGES
    senders_3d = jnp.pad(senders.astype(jnp.int32), (0, pad)).reshape(N_EBLK, 1, EB)
    receivers_3d = jnp.pad(receivers.astype(jnp.int32), (0, pad)).reshape(N_EBLK, 1, EB)
    ef_p = jnp.pad(edge_features, ((0, pad), (0, 0)))
    rad_p = jnp.pad(radial_embedding, ((0, pad), (0, 0)))
    W2r = jnp.concatenate(
        [
            W2[:, :D_FEAT],
            W2[:, D_FEAT:]
            .reshape(RADIAL_HIDDEN, D_FEAT, D_EDGE)
            .transpose(0, 2, 1)
            .reshape(RADIAL_HIDDEN, D_FEAT * D_EDGE),
        ],
        axis=1,
    )
    msgs = _sc_gather(node_feats, senders_3d)
    msgA = _dense_messages(msgs, ef_p, rad_p, W1, W2r, chunks=(0, 2))
    outA = _sc_scatter(msgA, receivers_3d, n_full=1, has_half=False)
    msgB = _dense_messages(msgs, ef_p, rad_p, W1, W2r, chunks=(1, 3, 4))
    outB = _sc_scatter(msgB, receivers_3d, n_full=1, has_half=True)
    blocked = jnp.concatenate(
        [outA[0], outB[0], outA[1], outB[1], outB[2] + outB[3]], axis=-1
    )
    direct = blocked[:, :D_FEAT]
    tp = (
        blocked[:, D_FEAT:]
        .reshape(N_NODES, D_EDGE, D_FEAT)
        .transpose(0, 2, 1)
        .reshape(N_NODES, D_FEAT * D_EDGE)
    )
    return jnp.concatenate([direct, tp], axis=-1)


# speedup vs baseline: 1.0192x; 1.0190x over previous
import math

import jax
import jax.numpy as jnp
from jax import lax
from jax.experimental import pallas as pl
from jax.experimental.pallas import tpu as pltpu
from jax.experimental.pallas import tpu_sc as plsc

N_NODES = 10000
N_EDGES = 160000
D_FEAT = 128
D_EDGE = 4
D_RAD = 8
RADIAL_HIDDEN = 8
D_MSG = D_FEAT + D_FEAT * D_EDGE
AVG_NEIGHBOURS = 16.0

_INV_SQRT_RAD = 1.0 / math.sqrt(float(D_RAD))
_W2_SCALE = (1.0 / math.sqrt(float(RADIAL_HIDDEN))) / math.sqrt(AVG_NEIGHBOURS)

NC = 2
NS = 16
NW = NC * NS

EB = 128
E_PAD = 163840
N_EBLK = E_PAD // EB
N_CHUNK = 5
D_CHUNK = 128

NODE_TILE = 624
NODE_TAIL = N_NODES - NODE_TILE * NS
ZCHUNK = 48

EDGE_BLOCK = 2048


def _sc_mesh():
    return plsc.VectorSubcoreMesh(core_axis_name="c", subcore_axis_name="s")


GDEPTH = 5


def _gather_body(nf_hbm, snd_hbm, out_hbm, idx_v, rows_v, *sems):
    c = lax.axis_index("c")
    s = lax.axis_index("s")
    wid = s * NC + c
    nblk = N_EBLK // NW
    row0 = wid * nblk
    pltpu.sync_copy(snd_hbm.at[pl.ds(row0, nblk)], idx_v)

    for par in range(GDEPTH):
        pltpu.make_async_copy(
            nf_hbm.at[idx_v.at[par, 0]], rows_v.at[par], sems[par]
        ).start()

    def body(g, carry):
        for par in range(GDEPTH):
            b = g * GDEPTH + par
            pltpu.make_async_copy(
                nf_hbm.at[idx_v.at[b, 0]], rows_v.at[par], sems[par]
            ).wait()
            pltpu.sync_copy(rows_v.at[par], out_hbm.at[pl.ds((row0 + b) * EB, EB)])

            @pl.when(g < nblk // GDEPTH - 1)
            def _next():
                pltpu.make_async_copy(
                    nf_hbm.at[idx_v.at[b + GDEPTH, 0]], rows_v.at[par], sems[par]
                ).start()

        return carry

    lax.fori_loop(0, nblk // GDEPTH, body, 0)


def _sc_gather(node_feats, senders_3d):
    return pl.kernel(
        _gather_body,
        out_type=jax.ShapeDtypeStruct((E_PAD, D_FEAT), jnp.float32),
        mesh=_sc_mesh(),
        scratch_types=[
            pltpu.VMEM((N_EBLK // NW, 1, EB), jnp.int32),
            pltpu.VMEM((GDEPTH, EB, D_FEAT), jnp.float32),
        ]
        + [pltpu.SemaphoreType.DMA] * GDEPTH,
    )(node_feats, senders_3d)


def _make_dense_body(chunks):
    nch = len(chunks)

    def _dense_body(msgs_ref, ef_ref, rad_ref, w1_ref, w2s_ref, out_ref):
        rad = rad_ref[...]
        h = jnp.dot(rad, w1_ref[...], preferred_element_type=jnp.float32)
        h = h * _INV_SQRT_RAD
        h = h * jax.nn.sigmoid(h)
        w = jnp.dot(h, w2s_ref[...], preferred_element_type=jnp.float32)
        w = w * _W2_SCALE
        m = msgs_ref[...]
        ef = ef_ref[...]
        for ci, ch in enumerate(chunks):
            wpart = w[:, ci * D_FEAT : (ci + 1) * D_FEAT]
            if ch == 0:
                out_ref[ci] = m * wpart
            else:
                out_ref[ci] = m * ef[:, ch - 1 : ch] * wpart

    return _dense_body


def _dense_messages(msgs, edge_features, radial_embedding, W1, W2r, chunks):
    w2s = jnp.concatenate(
        [W2r[:, ch * D_FEAT : (ch + 1) * D_FEAT] for ch in chunks], axis=1
    )
    nch = len(chunks)
    grid = (E_PAD // EDGE_BLOCK,)
    return pl.pallas_call(
        _make_dense_body(chunks),
        grid=grid,
        in_specs=[
            pl.BlockSpec((EDGE_BLOCK, D_FEAT), lambda i: (i, 0)),
            pl.BlockSpec((EDGE_BLOCK, D_EDGE), lambda i: (i, 0)),
            pl.BlockSpec((EDGE_BLOCK, D_RAD), lambda i: (i, 0)),
            pl.BlockSpec((D_RAD, RADIAL_HIDDEN), lambda i: (0, 0)),
            pl.BlockSpec((RADIAL_HIDDEN, nch * D_FEAT), lambda i: (0, 0)),
        ],
        out_specs=pl.BlockSpec((nch, EDGE_BLOCK, D_CHUNK), lambda i: (0, i, 0)),
        out_shape=jax.ShapeDtypeStruct((nch, E_PAD, D_CHUNK), jnp.float32),
    )(msgs, edge_features, radial_embedding, W1, w2s)


def _make_scatter_body(n_full, has_half):
    def _scatter_body(msg_hbm, recv_hbm, out_hbm, idx_v, buf_v, zbuf_v, acc_sh, sem0, sem1):
        c = lax.axis_index("c")
        s = lax.axis_index("s")
        nblk_full = N_EBLK // NS
        r0 = s * nblk_full
        sems = (sem0, sem1)

        def zero_body(i, carry):
            r = i // (D_CHUNK // 16)
            g = i % (D_CHUNK // 16)
            zbuf_v[r, pl.ds(g * 16, 16)] = jnp.zeros((16,), jnp.float32)
            return carry

        lax.fori_loop(0, ZCHUNK * (D_CHUNK // 16), zero_body, 0)

        pltpu.sync_copy(recv_hbm.at[pl.ds(r0, nblk_full)], idx_v)

        n_passes = n_full + (1 if has_half else 0)
        for p in range(n_passes):
            if p < n_full:
                chunk = c * n_full + p
                slot = chunk
                b_lo = 0
                nblk = nblk_full
            else:
                chunk = 2 * n_full
                slot = 2 * n_full + c
                b_lo = c * (nblk_full // 2)
                nblk = nblk_full // 2

            for k in range(NODE_TILE // ZCHUNK):
                pltpu.sync_copy(zbuf_v, acc_sh.at[pl.ds(s * NODE_TILE + k * ZCHUNK, ZCHUNK)])

            @pl.when(s == 0)
            def _zero_tail():
                pltpu.sync_copy(
                    zbuf_v.at[pl.ds(0, NODE_TAIL)],
                    acc_sh.at[pl.ds(NODE_TILE * NS, NODE_TAIL)],
                )

            plsc.subcore_barrier()

            for par in range(2):
                pltpu.make_async_copy(
                    msg_hbm.at[chunk, pl.ds((r0 + b_lo + par) * EB, EB)],
                    buf_v.at[par],
                    sems[par],
                ).start()

            def acc_body(g, carry):
                for par in range(2):
                    lb = b_lo + g * 2 + par
                    pltpu.make_async_copy(
                        msg_hbm.at[chunk, pl.ds((r0 + lb) * EB, EB)],
                        buf_v.at[par],
                        sems[par],
                    ).wait()
                    pltpu.sync_copy(buf_v.at[par], acc_sh.at[idx_v.at[lb, 0]], add=True)

                    @pl.when(g < nblk // 2 - 1)
                    def _next():
                        pltpu.make_async_copy(
                            msg_hbm.at[chunk, pl.ds((r0 + lb + 2) * EB, EB)],
                            buf_v.at[par],
                            sems[par],
                        ).start()

                return carry

            lax.fori_loop(0, nblk // 2, acc_body, 0)
            plsc.subcore_barrier()

            off = s * NODE_TILE
            pltpu.sync_copy(acc_sh.at[pl.ds(off, NODE_TILE)], out_hbm.at[slot, pl.ds(off, NODE_TILE)])

            @pl.when(s == 0)
            def _flush_tail():
                off_t = NODE_TILE * NS
                pltpu.sync_copy(
                    acc_sh.at[pl.ds(off_t, NODE_TAIL)],
                    out_hbm.at[slot, pl.ds(off_t, NODE_TAIL)],
                )

            plsc.subcore_barrier()

    return _scatter_body


def _sc_scatter(msg_parts, receivers_3d, n_full, has_half):
    n_parts = msg_parts.shape[0]
    n_slots = 2 * n_full + (2 if has_half else 0)
    assert n_parts == 2 * n_full + (1 if has_half else 0)
    return pl.kernel(
        _make_scatter_body(n_full, has_half),
        out_type=jax.ShapeDtypeStruct((n_slots, N_NODES, D_CHUNK), jnp.float32),
        mesh=_sc_mesh(),
        scratch_types=[
            pltpu.VMEM((N_EBLK // NS, 1, EB), jnp.int32),
            pltpu.VMEM((2, EB, D_CHUNK), jnp.float32),
            pltpu.VMEM((ZCHUNK, D_CHUNK), jnp.float32),
            pltpu.VMEM_SHARED((N_NODES, D_CHUNK), jnp.float32),
            pltpu.SemaphoreType.DMA,
            pltpu.SemaphoreType.DMA,
        ],
    )(msg_parts, receivers_3d)


def kernel(node_feats, edge_features, radial_embedding, senders, receivers, W1, W2):
    pad = E_PAD - N_EDGES
    senders_3d = jnp.pad(senders.astype(jnp.int32), (0, pad)).reshape(N_EBLK, 1, EB)
    receivers_3d = jnp.pad(receivers.astype(jnp.int32), (0, pad)).reshape(N_EBLK, 1, EB)
    ef_p = jnp.pad(edge_features, ((0, pad), (0, 0)))
    rad_p = jnp.pad(radial_embedding, ((0, pad), (0, 0)))
    W2r = jnp.concatenate(
        [
            W2[:, :D_FEAT],
            W2[:, D_FEAT:]
            .reshape(RADIAL_HIDDEN, D_FEAT, D_EDGE)
            .transpose(0, 2, 1)
            .reshape(RADIAL_HIDDEN, D_FEAT * D_EDGE),
        ],
        axis=1,
    )
    msgs = _sc_gather(node_feats, senders_3d)
    msg5 = _dense_messages(msgs, ef_p, rad_p, W1, W2r, chunks=(0, 1, 2, 3, 4))
    out6 = _sc_scatter(msg5, receivers_3d, n_full=2, has_half=True)
    blocked = jnp.concatenate(
        [out6[0], out6[1], out6[2], out6[3], out6[4] + out6[5]], axis=-1
    )
    direct = blocked[:, :D_FEAT]
    tp = (
        blocked[:, D_FEAT:]
        .reshape(N_NODES, D_EDGE, D_FEAT)
        .transpose(0, 2, 1)
        .reshape(N_NODES, D_FEAT * D_EDGE)
    )
    return jnp.concatenate([direct, tp], axis=-1)
